# Initial kernel scaffold; baseline (speedup 1.0000x reference)
import jax
import jax.numpy as jnp
from jax.experimental import pallas as pl


def kernel(notes, onsets, durations, x_lengths, table):
    raise NotImplementedError("write your pallas kernel here")


# speedup vs baseline: 3.5286x; 3.5286x over previous
import jax
import jax.numpy as jnp
from jax import lax
from jax.experimental import pallas as pl
from jax.experimental.pallas import tpu as pltpu
from jax.experimental.pallas import tpu_sc as plsc

B, L = 4096, 200
VOCAB, EDIM = 91, 8
N = B * L
OUT_D = EDIM + 2
NC, NS = 2, 16
NW = NC * NS
PER_W = N // NW
CHUNK = 3200
NCHUNK = PER_W // CHUNK
GROUPS = CHUNK // 16
TBL_PAD = 96


def _body(notes_hbm, ons_hbm, dur_hbm, tbl_hbm, out_hbm,
          tbl_v, notes_v, ons_v, dur_v, out_v):
    wid = lax.axis_index("s") * NC + lax.axis_index("c")
    pltpu.sync_copy(tbl_hbm, tbl_v)
    iota10 = lax.iota(jnp.int32, 16) * OUT_D

    for c in range(NCHUNK):
        base = wid * PER_W + c * CHUNK
        pltpu.sync_copy(notes_hbm.at[pl.ds(base, CHUNK)], notes_v)
        pltpu.sync_copy(ons_hbm.at[pl.ds(base, CHUNK)], ons_v)
        pltpu.sync_copy(dur_hbm.at[pl.ds(base, CHUNK)], dur_v)

        def grp(i, carry):
            e = i * 16
            n = notes_v[pl.ds(e, 16)]
            o = ons_v[pl.ds(e, 16)]
            d = dur_v[pl.ds(e, 16)]
            gb = n * EDIM
            sidx = iota10 + i * (16 * OUT_D)
            for j in range(EDIM):
                colv = plsc.load_gather(tbl_v, [gb + j])
                plsc.store_scatter(out_v, [sidx + j], colv)
            plsc.store_scatter(out_v, [sidx + EDIM], o)
            plsc.store_scatter(out_v, [sidx + EDIM + 1], d)
            return carry

        lax.fori_loop(0, GROUPS, grp, 0)
        pltpu.sync_copy(out_v, out_hbm.at[pl.ds(base * OUT_D, CHUNK * OUT_D)])


def kernel(notes, onsets, durations, x_lengths, table):
    del x_lengths
    notes_f = notes.reshape(N).astype(jnp.int32)
    ons_f = onsets.reshape(N)
    dur_f = durations.reshape(N)
    tbl_pad = jnp.zeros((TBL_PAD, EDIM), jnp.float32).at[:VOCAB].set(table)
    tbl_flat = tbl_pad.reshape(TBL_PAD * EDIM)

    mesh = plsc.VectorSubcoreMesh(core_axis_name="c", subcore_axis_name="s")
    out = pl.kernel(
        _body,
        mesh=mesh,
        compiler_params=pltpu.CompilerParams(needs_layout_passes=False),
        out_type=jax.ShapeDtypeStruct((N * OUT_D,), jnp.float32),
        scratch_types=[
            pltpu.VMEM((TBL_PAD * EDIM,), jnp.float32),
            pltpu.VMEM((CHUNK,), jnp.int32),
            pltpu.VMEM((CHUNK,), jnp.float32),
            pltpu.VMEM((CHUNK,), jnp.float32),
            pltpu.VMEM((CHUNK * OUT_D,), jnp.float32),
        ],
    )(notes_f, ons_f, dur_f, tbl_flat)
    return out.reshape(B, L, OUT_D)


# speedup vs baseline: 27.5389x; 7.8045x over previous
import jax
import jax.numpy as jnp
from jax import lax
from jax.experimental import pallas as pl
from jax.experimental.pallas import tpu as pltpu
from jax.experimental.pallas import tpu_sc as plsc

B, L = 4096, 200
VOCAB, EDIM = 91, 8
OUT_D = EDIM + 2
NC, NS = 2, 16
NW = NC * NS
NLT = L // 8
NLT_CHUNK = 5
NCHUNK = NLT // NLT_CHUNK
ROWS = NLT_CHUNK * 8
GROUPS = 128 // 16


def _body(notes_hbm, ons_hbm, dur_hbm, tbl_hbm, out_hbm,
          tbl_v, notes_v, ons_v, dur_v, out_v):
    wid = lax.axis_index("s") * NC + lax.axis_index("c")
    pltpu.sync_copy(tbl_hbm, tbl_v)
    jj = [jnp.full((16,), d, jnp.int32) for d in range(EDIM)]

    for c in range(NCHUNK):
        lt0 = c * NLT_CHUNK
        pltpu.sync_copy(notes_hbm.at[pl.ds(lt0, NLT_CHUNK), :, wid], notes_v)
        pltpu.sync_copy(ons_hbm.at[pl.ds(lt0, NLT_CHUNK), :, wid], ons_v)
        pltpu.sync_copy(dur_hbm.at[pl.ds(lt0, NLT_CHUNK), :, wid], dur_v)

        def row(r, carry):
            lt_i = r // 8
            li = r % 8

            def grp(q, carry2):
                s = pl.ds(q * 16, 16)
                n = notes_v[lt_i, li, s]
                for d in range(EDIM):
                    out_v[d, lt_i, li, s] = plsc.load_gather(tbl_v, [jj[d], n])
                return carry2

            lax.fori_loop(0, GROUPS, grp, 0)
            return carry

        lax.fori_loop(0, ROWS, row, 0)

        pltpu.sync_copy(out_v, out_hbm.at[pl.ds(0, EDIM), pl.ds(lt0, NLT_CHUNK), wid])
        pltpu.sync_copy(ons_v, out_hbm.at[EDIM, pl.ds(lt0, NLT_CHUNK), wid])
        pltpu.sync_copy(dur_v, out_hbm.at[EDIM + 1, pl.ds(lt0, NLT_CHUNK), wid])


def kernel(notes, onsets, durations, x_lengths, table):
    del x_lengths
    notes_t = jnp.transpose(notes, (1, 2, 0)).reshape(NLT, 8, NW, 128)
    ons_t = jnp.transpose(onsets, (1, 2, 0)).reshape(NLT, 8, NW, 128)
    dur_t = jnp.transpose(durations, (1, 2, 0)).reshape(NLT, 8, NW, 128)
    tbl_t = jnp.transpose(table)

    mesh = plsc.VectorSubcoreMesh(core_axis_name="c", subcore_axis_name="s")
    y = pl.kernel(
        _body,
        mesh=mesh,
        compiler_params=pltpu.CompilerParams(needs_layout_passes=False),
        out_type=jax.ShapeDtypeStruct((OUT_D, NLT, NW, 8, 128), jnp.float32),
        scratch_types=[
            pltpu.VMEM((EDIM, VOCAB), jnp.float32),
            pltpu.VMEM((NLT_CHUNK, 8, 128), jnp.int32),
            pltpu.VMEM((NLT_CHUNK, 8, 128), jnp.float32),
            pltpu.VMEM((NLT_CHUNK, 8, 128), jnp.float32),
            pltpu.VMEM((EDIM, NLT_CHUNK, 8, 128), jnp.float32),
        ],
    )(notes_t, ons_t, dur_t, tbl_t)
    return jnp.transpose(y, (2, 4, 1, 3, 0)).reshape(B, L, OUT_D)


# speedup vs baseline: 31.3448x; 1.1382x over previous
import jax
import jax.numpy as jnp
from jax import lax
from jax.experimental import pallas as pl
from jax.experimental.pallas import tpu as pltpu
from jax.experimental.pallas import tpu_sc as plsc

B, L = 4096, 200
VOCAB, EDIM = 91, 8
OUT_D = EDIM + 2
NC, NS = 2, 16
NW = NC * NS
NLT = L // 8
NLT_CHUNK = 5
NCHUNK = NLT // NLT_CHUNK
ROWS = NLT_CHUNK * 8
GROUPS = 128 // 16


def _body(notes_hbm, ons_hbm, dur_hbm, tbl_hbm, out_hbm,
          tbl_v, notes_v0, notes_v1, out_v0, out_v1,
          sem_in0, sem_in1, sem_out0, sem_out1):
    wid = lax.axis_index("s") * NC + lax.axis_index("c")
    pltpu.sync_copy(tbl_hbm, tbl_v)
    jj = [jnp.full((16,), d, jnp.int32) for d in range(EDIM)]

    notes_bufs = (notes_v0, notes_v1)
    out_bufs = (out_v0, out_v1)
    sems_in = (sem_in0, sem_in1)
    sems_out = (sem_out0, sem_out1)

    def start_in(c):
        s = c % 2
        sl = pl.ds(c * NLT_CHUNK, NLT_CHUNK)
        return [
            pltpu.async_copy(notes_hbm.at[sl, :, wid], notes_bufs[s], sems_in[s]),
            pltpu.async_copy(ons_hbm.at[sl, :, wid], out_bufs[s].at[EDIM], sems_in[s]),
            pltpu.async_copy(dur_hbm.at[sl, :, wid], out_bufs[s].at[EDIM + 1], sems_in[s]),
        ]

    def start_out(c):
        s = c % 2
        sl = pl.ds(c * NLT_CHUNK, NLT_CHUNK)
        return pltpu.async_copy(out_bufs[s], out_hbm.at[:, sl, wid], sems_out[s])

    def compute(c):
        s = c % 2
        notes_s = notes_bufs[s]
        out_s = out_bufs[s]

        def row(r, carry):
            lt_i = r // 8
            li = r % 8

            def grp(q, c2):
                sl = pl.ds(q * 16, 16)
                n = notes_s[lt_i, li, sl]
                for d in range(EDIM):
                    out_s[d, lt_i, li, sl] = plsc.load_gather(tbl_v, [jj[d], n])
                return c2

            lax.fori_loop(0, GROUPS, grp, 0)
            return carry

        lax.fori_loop(0, ROWS, row, 0)

    cps_in = {0: start_in(0), 1: start_in(1)}
    cps_out = {}
    for c in range(NCHUNK):
        for cp in cps_in.pop(c):
            cp.wait()
        compute(c)
        if c >= 1 and c + 1 < NCHUNK:
            cps_out.pop(c - 1).wait()
            cps_in[c + 1] = start_in(c + 1)
        cps_out[c] = start_out(c)
    for c in sorted(cps_out):
        cps_out.pop(c).wait()


def kernel(notes, onsets, durations, x_lengths, table):
    del x_lengths
    notes_t = jnp.transpose(notes, (1, 2, 0)).reshape(NLT, 8, NW, 128)
    ons_t = jnp.transpose(onsets, (1, 2, 0)).reshape(NLT, 8, NW, 128)
    dur_t = jnp.transpose(durations, (1, 2, 0)).reshape(NLT, 8, NW, 128)
    tbl_t = jnp.transpose(table)

    mesh = plsc.VectorSubcoreMesh(core_axis_name="c", subcore_axis_name="s")
    y = pl.kernel(
        _body,
        mesh=mesh,
        compiler_params=pltpu.CompilerParams(needs_layout_passes=False),
        out_type=jax.ShapeDtypeStruct((OUT_D, NLT, NW, 8, 128), jnp.float32),
        scratch_types=[
            pltpu.VMEM((EDIM, VOCAB), jnp.float32),
            pltpu.VMEM((NLT_CHUNK, 8, 128), jnp.int32),
            pltpu.VMEM((NLT_CHUNK, 8, 128), jnp.int32),
            pltpu.VMEM((OUT_D, NLT_CHUNK, 8, 128), jnp.float32),
            pltpu.VMEM((OUT_D, NLT_CHUNK, 8, 128), jnp.float32),
            pltpu.SemaphoreType.DMA,
            pltpu.SemaphoreType.DMA,
            pltpu.SemaphoreType.DMA,
            pltpu.SemaphoreType.DMA,
        ],
    )(notes_t, ons_t, dur_t, tbl_t)
    return jnp.transpose(y, (2, 4, 1, 3, 0)).reshape(B, L, OUT_D)
